# Initial kernel scaffold; baseline (speedup 1.0000x reference)
import jax
import jax.numpy as jnp
from jax.experimental import pallas as pl


def kernel(cons_features, edge_indices, vars_features, W_ce, b_ce, W_ve, b_ve, W_j1, b_j1, W_j2, b_j2, W_cr, b_cr, W_vr, b_vr, W_o1, b_o1, W_o2, b_o2, W_o3, b_o3):
    raise NotImplementedError("write your pallas kernel here")


# speedup vs baseline: 3.4292x; 3.4292x over previous
import functools

import jax
import jax.numpy as jnp
from jax import lax
from jax.experimental import pallas as pl
from jax.experimental.pallas import tpu as pltpu
from jax.experimental.pallas import tpu_sc as plsc

N = 100000
E = 1600000
EMB = 32
H = 16

NC = 2
NS = 16
NW = NC * NS

EW = E // NW
GCH = 1000
GNCH = EW // GCH

ES = E // NS
SCH = 1000
SNCH = ES // SCH

NR = N // NS
ZN = NR // SCH
ZT = NR - ZN * SCH

BN = 4000
BE = 8000

_f32 = jnp.float32


def _sc_gather_body(a_hbm, b_hbm, src_hbm, dst_hbm, out_hbm,
                    idx_a, idx_b, buf_a, buf_b, sem_a, sem_b):
    cid = lax.axis_index("c")
    sid = lax.axis_index("s")
    wid = cid * NS + sid

    def chunk(k, carry):
        base = pl.multiple_of(wid * EW + k * GCH, 8)
        pltpu.sync_copy(src_hbm.at[pl.ds(base, GCH)], idx_a)
        pltpu.sync_copy(dst_hbm.at[pl.ds(base, GCH)], idx_b)
        cp_a = pltpu.async_copy(a_hbm.at[idx_a], buf_a, sem_a)
        cp_b = pltpu.async_copy(b_hbm.at[idx_b], buf_b, sem_b)
        cp_a.wait()
        cp_b.wait()

        def add_row(i, c2):
            buf_a[i, pl.ds(0, 16)] += buf_b[i, pl.ds(0, 16)]
            buf_a[i, pl.ds(16, 16)] += buf_b[i, pl.ds(16, 16)]
            return c2

        lax.fori_loop(0, GCH, add_row, 0)
        pltpu.sync_copy(buf_a, out_hbm.at[pl.ds(base, GCH)])
        return carry

    lax.fori_loop(0, GNCH, chunk, 0)


_sc_gather = pl.kernel(
    _sc_gather_body,
    out_type=jax.ShapeDtypeStruct((E, EMB), _f32),
    compiler_params=pltpu.CompilerParams(use_tc_tiling_on_sc=False),
    mesh=plsc.VectorSubcoreMesh(core_axis_name="c", subcore_axis_name="s"),
    scratch_types=[
        pltpu.VMEM((GCH,), jnp.int32),
        pltpu.VMEM((GCH,), jnp.int32),
        pltpu.VMEM((GCH, EMB), _f32),
        pltpu.VMEM((GCH, EMB), _f32),
        pltpu.SemaphoreType.DMA,
        pltpu.SemaphoreType.DMA,
    ],
)


def _sc_scatter_body(h2_lo, h2_hi, idx_hbm, out_lo, out_hi,
                     idx_v, rows_v, acc):
    cid = lax.axis_index("c")
    sid = lax.axis_index("s")

    def zrow(i, c2):
        rows_v[i, :] = jnp.zeros((16,), _f32)
        return c2

    lax.fori_loop(0, SCH, zrow, 0)

    def zcp(j, c2):
        pltpu.sync_copy(rows_v, acc.at[pl.ds(sid * NR + j * SCH, SCH)])
        return c2

    lax.fori_loop(0, ZN, zcp, 0)
    pltpu.sync_copy(rows_v.at[pl.ds(0, ZT)],
                    acc.at[pl.ds(sid * NR + ZN * SCH, ZT)])
    plsc.subcore_barrier()

    def run_chunks(h2):
        def chunk(k, carry):
            base = pl.multiple_of(sid * ES + k * SCH, 8)
            pltpu.sync_copy(idx_hbm.at[pl.ds(base, SCH)], idx_v)
            pltpu.sync_copy(h2.at[pl.ds(base, SCH)], rows_v)
            pltpu.sync_copy(rows_v, acc.at[idx_v], add=True)
            return carry
        lax.fori_loop(0, SNCH, chunk, 0)

    @pl.when(cid == 0)
    def _():
        run_chunks(h2_lo)

    @pl.when(cid == 1)
    def _():
        run_chunks(h2_hi)

    plsc.subcore_barrier()
    r0 = sid * NR

    @pl.when(cid == 0)
    def _():
        pltpu.sync_copy(acc.at[pl.ds(r0, NR)], out_lo.at[pl.ds(r0, NR)])

    @pl.when(cid == 1)
    def _():
        pltpu.sync_copy(acc.at[pl.ds(r0, NR)], out_hi.at[pl.ds(r0, NR)])


_sc_scatter = pl.kernel(
    _sc_scatter_body,
    out_type=(jax.ShapeDtypeStruct((N, H), _f32),
              jax.ShapeDtypeStruct((N, H), _f32)),
    compiler_params=pltpu.CompilerParams(use_tc_tiling_on_sc=False),
    mesh=plsc.VectorSubcoreMesh(core_axis_name="c", subcore_axis_name="s"),
    scratch_types=[
        pltpu.VMEM((SCH,), jnp.int32),
        pltpu.VMEM((SCH, H), _f32),
        pltpu.VMEM_SHARED((N, H), _f32),
    ],
)


def _full(shape):
    return pl.BlockSpec(shape, lambda i: tuple(0 for _ in shape))


def _tc_pre_body(cf, vf, wce, bce, wve, bve, wj1,
                 c_o, v_o, a_o, b_o):
    c = bce[...][None, :]
    for k in range(2):
        c = c + cf[...][:, k:k + 1] * wce[...][k:k + 1, :]
    c = jnp.maximum(c, 0.0)
    v = bve[...][None, :]
    for k in range(9):
        v = v + vf[...][:, k:k + 1] * wve[...][k:k + 1, :]
    v = jnp.maximum(v, 0.0)
    c_o[...] = c
    v_o[...] = v
    wj1v = wj1[...]
    a_o[...] = jnp.dot(c, wj1v[:EMB, :], preferred_element_type=_f32)
    b_o[...] = jnp.dot(v, wj1v[EMB:, :], preferred_element_type=_f32)


def _tc_pre(cf, vf, wce, bce, wve, bve, wj1):
    return pl.pallas_call(
        _tc_pre_body,
        grid=(N // BN,),
        in_specs=[
            pl.BlockSpec((BN, 2), lambda i: (i, 0)),
            pl.BlockSpec((BN, 9), lambda i: (i, 0)),
            _full((2, EMB)), _full((EMB,)),
            _full((9, EMB)), _full((EMB,)),
            _full((2 * EMB, EMB)),
        ],
        out_specs=[pl.BlockSpec((BN, EMB), lambda i: (i, 0))] * 4,
        out_shape=[jax.ShapeDtypeStruct((N, EMB), _f32)] * 4,
    )(cf, vf, wce, bce, wve, bve, wj1)


def _tc_mlp_body(x, bj1, wj2, bj2, lo_o, hi_o):
    h1 = jnp.maximum(x[...] + bj1[...][None, :], 0.0)
    h2 = jnp.dot(h1, wj2[...], preferred_element_type=_f32)
    h2 = jnp.maximum(h2 + bj2[...][None, :], 0.0)
    lo_o[...] = h2[:, :H]
    hi_o[...] = h2[:, H:]


def _tc_mlp(h1pre, bj1, wj2, bj2):
    return pl.pallas_call(
        _tc_mlp_body,
        grid=(E // BE,),
        in_specs=[
            pl.BlockSpec((BE, EMB), lambda i: (i, 0)),
            _full((EMB,)), _full((EMB, EMB)), _full((EMB,)),
        ],
        out_specs=[pl.BlockSpec((BE, H), lambda i: (i, 0))] * 2,
        out_shape=[jax.ShapeDtypeStruct((E, H), _f32)] * 2,
    )(h1pre, bj1, wj2, bj2)


def _tc_update_body(alo, ahi, c, wcr, bcr, wj1, a2_o):
    agg_lo = alo[...]
    agg_hi = ahi[...]
    wcrv = wcr[...]
    x = jnp.dot(agg_lo, wcrv[:H, :], preferred_element_type=_f32)
    x = x + jnp.dot(agg_hi, wcrv[H:EMB, :], preferred_element_type=_f32)
    x = x + jnp.dot(c[...], wcrv[EMB:, :], preferred_element_type=_f32)
    x = jnp.maximum(x + bcr[...][None, :], 0.0)
    a2_o[...] = jnp.dot(x, wj1[...][:EMB, :], preferred_element_type=_f32)


def _tc_update(alo, ahi, c, wcr, bcr, wj1):
    return pl.pallas_call(
        _tc_update_body,
        grid=(N // BN,),
        in_specs=[
            pl.BlockSpec((BN, H), lambda i: (i, 0)),
            pl.BlockSpec((BN, H), lambda i: (i, 0)),
            pl.BlockSpec((BN, EMB), lambda i: (i, 0)),
            _full((2 * EMB, EMB)), _full((EMB,)), _full((2 * EMB, EMB)),
        ],
        out_specs=pl.BlockSpec((BN, EMB), lambda i: (i, 0)),
        out_shape=jax.ShapeDtypeStruct((N, EMB), _f32),
    )(alo, ahi, c, wcr, bcr, wj1)


def _tc_out_body(alo, ahi, v, wvr, bvr, wo1, bo1, wo2, bo2, wo3, bo3, o):
    wvrv = wvr[...]
    x = jnp.dot(alo[...], wvrv[:H, :], preferred_element_type=_f32)
    x = x + jnp.dot(ahi[...], wvrv[H:EMB, :], preferred_element_type=_f32)
    x = x + jnp.dot(v[...], wvrv[EMB:, :], preferred_element_type=_f32)
    x = jnp.maximum(x + bvr[...][None, :], 0.0)
    x = jnp.maximum(jnp.dot(x, wo1[...], preferred_element_type=_f32)
                    + bo1[...][None, :], 0.0)
    x = jnp.maximum(jnp.dot(x, wo2[...], preferred_element_type=_f32)
                    + bo2[...][None, :], 0.0)
    o[...] = jnp.dot(x, wo3[...], preferred_element_type=_f32) + bo3[...][None, :]


def _tc_out(alo, ahi, v, wvr, bvr, wo1, bo1, wo2, bo2, wo3, bo3):
    return pl.pallas_call(
        _tc_out_body,
        grid=(N // BN,),
        in_specs=[
            pl.BlockSpec((BN, H), lambda i: (i, 0)),
            pl.BlockSpec((BN, H), lambda i: (i, 0)),
            pl.BlockSpec((BN, EMB), lambda i: (i, 0)),
            _full((2 * EMB, EMB)), _full((EMB,)),
            _full((EMB, EMB)), _full((EMB,)),
            _full((EMB, EMB)), _full((EMB,)),
            _full((EMB, 1)), _full((1,)),
        ],
        out_specs=pl.BlockSpec((BN, 1), lambda i: (i, 0)),
        out_shape=jax.ShapeDtypeStruct((N, 1), _f32),
    )(alo, ahi, v, wvr, bvr, wo1, bo1, wo2, bo2, wo3, bo3)


def kernel(cons_features, edge_indices, vars_features,
           W_ce, b_ce, W_ve, b_ve, W_j1, b_j1, W_j2, b_j2,
           W_cr, b_cr, W_vr, b_vr, W_o1, b_o1, W_o2, b_o2, W_o3, b_o3):
    src = edge_indices[0].astype(jnp.int32)
    dst = edge_indices[1].astype(jnp.int32)

    c, v, a1, b = _tc_pre(cons_features, vars_features,
                          W_ce, b_ce, W_ve, b_ve, W_j1)

    h1pre = _sc_gather(a1, b, src, dst)
    h2_lo, h2_hi = _tc_mlp(h1pre, b_j1, W_j2, b_j2)
    agg_lo, agg_hi = _sc_scatter(h2_lo, h2_hi, src)

    a2 = _tc_update(agg_lo, agg_hi, c, W_cr, b_cr, W_j1)

    h1pre2 = _sc_gather(a2, b, src, dst)
    h2b_lo, h2b_hi = _tc_mlp(h1pre2, b_j1, W_j2, b_j2)
    agg2_lo, agg2_hi = _sc_scatter(h2b_lo, h2b_hi, dst)

    return _tc_out(agg2_lo, agg2_hi, v, W_vr, b_vr,
                   W_o1, b_o1, W_o2, b_o2, W_o3, b_o3)


# speedup vs baseline: 11.4796x; 3.3476x over previous
import jax
import jax.numpy as jnp
from jax import lax
from jax.experimental import pallas as pl
from jax.experimental.pallas import tpu as pltpu
from jax.experimental.pallas import tpu_sc as plsc

N = 100000
E = 1600000
EH = E // 2
EMB = 32
H = 16
G = 4

NC = 2
NS = 16
NW = NC * NS

EW = EH // NW
GCH = 1000
GNCH = EW // GCH

ES = EH // NS
SCH = 400
SNCH = ES // SCH

NR = N // NS
ZN = NR // SCH
ZT = NR - ZN * SCH

NG = N // G
EHG = EH // G
BN = 5000
BE = 8000

_f32 = jnp.float32
_CP = pltpu.CompilerParams(use_tc_tiling_on_sc=False)
_MESH = dict(core_axis_name="c", subcore_axis_name="s")


def _make_gather2(e0):
    def body(a_hbm, b_hbm, src_hbm, dst_hbm, outa_hbm, outb_hbm,
             idx_a, idx_b, buf_a, buf_b, gsem, wsem):
        cid = lax.axis_index("c")
        sid = lax.axis_index("s")
        w0 = (cid * NS + sid) * EW

        def load_fire(k, s):
            base = pl.multiple_of(w0 + k * GCH, 8)
            pltpu.sync_copy(src_hbm.at[pl.ds(e0 + base, GCH)], idx_a)
            pltpu.sync_copy(dst_hbm.at[pl.ds(e0 + base, GCH)], idx_b)
            pltpu.async_copy(a_hbm.at[idx_a], buf_a[s], gsem[s])
            pltpu.async_copy(b_hbm.at[idx_b], buf_b[s], gsem[s])

        def wait_gather(s):
            pltpu.make_async_copy(a_hbm.at[idx_a], buf_a[s], gsem[s]).wait()
            pltpu.make_async_copy(b_hbm.at[idx_b], buf_b[s], gsem[s]).wait()

        def fire_write(k, s):
            base = pl.multiple_of(w0 + k * GCH, 8)
            pltpu.async_copy(buf_a[s], outa_hbm.at[pl.ds(base, GCH)], wsem[s])
            pltpu.async_copy(buf_b[s], outb_hbm.at[pl.ds(base, GCH)], wsem[s])

        def wait_write(k, s):
            base = pl.multiple_of(w0 + k * GCH, 8)
            pltpu.make_async_copy(buf_a[s], outa_hbm.at[pl.ds(base, GCH)],
                                  wsem[s]).wait()
            pltpu.make_async_copy(buf_b[s], outb_hbm.at[pl.ds(base, GCH)],
                                  wsem[s]).wait()

        load_fire(0, 0)

        def pair(p, carry):
            k = 2 * p
            wait_gather(0)
            fire_write(k, 0)

            @pl.when(p > 0)
            def _():
                wait_write(k - 1, 1)

            load_fire(k + 1, 1)
            wait_gather(1)
            fire_write(k + 1, 1)
            wait_write(k, 0)
            load_fire(k + 2, 0)
            return carry

        lax.fori_loop(0, GNCH // 2, pair, 0)
        kl = GNCH - 1
        wait_gather(0)
        fire_write(kl, 0)
        wait_write(kl - 1, 1)
        wait_write(kl, 0)

    return pl.kernel(
        body,
        out_type=(jax.ShapeDtypeStruct((EH, EMB), _f32),
                  jax.ShapeDtypeStruct((EH, EMB), _f32)),
        compiler_params=_CP,
        mesh=plsc.VectorSubcoreMesh(**_MESH),
        scratch_types=[
            pltpu.VMEM((GCH,), jnp.int32),
            pltpu.VMEM((GCH,), jnp.int32),
            [pltpu.VMEM((GCH, EMB), _f32)] * 2,
            [pltpu.VMEM((GCH, EMB), _f32)] * 2,
            [pltpu.SemaphoreType.DMA] * 2,
            [pltpu.SemaphoreType.DMA] * 2,
        ],
    )


def _make_gather1(e0):
    def body(a_hbm, idx_hbm, out_hbm, idx_a, buf_a, gsem, wsem):
        cid = lax.axis_index("c")
        sid = lax.axis_index("s")
        w0 = (cid * NS + sid) * EW

        def load_fire(k, s):
            base = pl.multiple_of(w0 + k * GCH, 8)
            pltpu.sync_copy(idx_hbm.at[pl.ds(e0 + base, GCH)], idx_a)
            pltpu.async_copy(a_hbm.at[idx_a], buf_a[s], gsem[s])

        def wait_gather(s):
            pltpu.make_async_copy(a_hbm.at[idx_a], buf_a[s], gsem[s]).wait()

        def fire_write(k, s):
            base = pl.multiple_of(w0 + k * GCH, 8)
            pltpu.async_copy(buf_a[s], out_hbm.at[pl.ds(base, GCH)], wsem[s])

        def wait_write(k, s):
            base = pl.multiple_of(w0 + k * GCH, 8)
            pltpu.make_async_copy(buf_a[s], out_hbm.at[pl.ds(base, GCH)],
                                  wsem[s]).wait()

        load_fire(0, 0)

        def pair(p, carry):
            k = 2 * p
            wait_gather(0)
            fire_write(k, 0)

            @pl.when(p > 0)
            def _():
                wait_write(k - 1, 1)

            load_fire(k + 1, 1)
            wait_gather(1)
            fire_write(k + 1, 1)
            wait_write(k, 0)
            load_fire(k + 2, 0)
            return carry

        lax.fori_loop(0, GNCH // 2, pair, 0)
        kl = GNCH - 1
        wait_gather(0)
        fire_write(kl, 0)
        wait_write(kl - 1, 1)
        wait_write(kl, 0)

    return pl.kernel(
        body,
        out_type=jax.ShapeDtypeStruct((EH, EMB), _f32),
        compiler_params=_CP,
        mesh=plsc.VectorSubcoreMesh(**_MESH),
        scratch_types=[
            pltpu.VMEM((GCH,), jnp.int32),
            [pltpu.VMEM((GCH, EMB), _f32)] * 2,
            [pltpu.SemaphoreType.DMA] * 2,
            [pltpu.SemaphoreType.DMA] * 2,
        ],
    )


def _make_scatter(e0):
    def body(h2_hbm, idx_hbm, out_hbm, idx_v, rows_v, rsem, acc):
        cid = lax.axis_index("c")
        sid = lax.axis_index("s")
        s0 = sid * ES

        def zrow(i, c2):
            rows_v[0][i, :] = jnp.zeros((16,), _f32)
            return c2

        lax.fori_loop(0, SCH, zrow, 0)

        def zcp(j, c2):
            pltpu.sync_copy(rows_v[0], acc.at[pl.ds(sid * NR + j * SCH, SCH)])
            return c2

        lax.fori_loop(0, ZN, zcp, 0)
        pltpu.sync_copy(rows_v[0].at[pl.ds(0, ZT)],
                        acc.at[pl.ds(sid * NR + ZN * SCH, ZT)])
        plsc.subcore_barrier()

        def run_chunks(col):
            def load_fire(k, s):
                base = pl.multiple_of(s0 + k * SCH, 8)
                pltpu.sync_copy(idx_hbm.at[pl.ds(e0 + base, SCH)], idx_v[s])
                pltpu.async_copy(h2_hbm.at[pl.ds(base, SCH), pl.ds(col, H)],
                                 rows_v[s], rsem[s])

            def wait_read(k, s):
                base = pl.multiple_of(s0 + k * SCH, 8)
                pltpu.make_async_copy(
                    h2_hbm.at[pl.ds(base, SCH), pl.ds(col, H)],
                    rows_v[s], rsem[s]).wait()

            load_fire(0, 0)

            def pair(p, carry):
                k = 2 * p
                wait_read(k, 0)
                load_fire(k + 1, 1)
                pltpu.sync_copy(rows_v[0], acc.at[idx_v[0]], add=True)
                wait_read(k + 1, 1)
                load_fire(k + 2, 0)
                pltpu.sync_copy(rows_v[1], acc.at[idx_v[1]], add=True)
                return carry

            lax.fori_loop(0, SNCH // 2, pair, 0)
            wait_read(SNCH - 1, 0)
            pltpu.sync_copy(rows_v[0], acc.at[idx_v[0]], add=True)

        @pl.when(cid == 0)
        def _():
            run_chunks(0)

        @pl.when(cid == 1)
        def _():
            run_chunks(H)

        plsc.subcore_barrier()
        r0 = sid * NR

        @pl.when(cid == 0)
        def _():
            pltpu.sync_copy(acc.at[pl.ds(r0, NR)],
                            out_hbm.at[pl.ds(r0, NR), pl.ds(0, H)])

        @pl.when(cid == 1)
        def _():
            pltpu.sync_copy(acc.at[pl.ds(r0, NR)],
                            out_hbm.at[pl.ds(r0, NR), pl.ds(H, H)])

    return pl.kernel(
        body,
        out_type=jax.ShapeDtypeStruct((N, EMB), _f32),
        compiler_params=_CP,
        mesh=plsc.VectorSubcoreMesh(**_MESH),
        scratch_types=[
            [pltpu.VMEM((SCH,), jnp.int32)] * 2,
            [pltpu.VMEM((SCH, H), _f32)] * 2,
            [pltpu.SemaphoreType.DMA] * 2,
            pltpu.VMEM_SHARED((N, H), _f32),
        ],
    )


_sc_gather2_h0 = _make_gather2(0)
_sc_gather2_h1 = _make_gather2(EH)
_sc_gather1_h0 = _make_gather1(0)
_sc_gather1_h1 = _make_gather1(EH)
_sc_scatter_h0 = _make_scatter(0)
_sc_scatter_h1 = _make_scatter(EH)


def _full(shape):
    return pl.BlockSpec(shape, lambda i: tuple(0 for _ in shape))


def _dot(a, b):
    return jnp.dot(a, b, preferred_element_type=_f32)


def _tc_pre_body(cf, vf, wce, bce, wve, bve, wj1a, wj1b,
                 c_o, v_o, a_o, b_o):
    c = jnp.maximum(_dot(cf[...], wce[...]) + bce[...][None, :], 0.0)
    v = jnp.maximum(_dot(vf[...], wve[...]) + bve[...][None, :], 0.0)
    c_o[...] = c
    v_o[...] = v
    a_o[...] = _dot(c, wj1a[...])
    b_o[...] = _dot(v, wj1b[...])


def _tc_pre(cf_g, vf_g, wce, bce, wve, bve, wj1a, wj1b):
    return pl.pallas_call(
        _tc_pre_body,
        grid=(NG // BN,),
        in_specs=[
            pl.BlockSpec((BN, 2 * G), lambda i: (i, 0)),
            pl.BlockSpec((BN, 9 * G), lambda i: (i, 0)),
            _full((2 * G, 32 * G)), _full((32 * G,)),
            _full((9 * G, 32 * G)), _full((32 * G,)),
            _full((32 * G, 32 * G)), _full((32 * G, 32 * G)),
        ],
        out_specs=[pl.BlockSpec((BN, 32 * G), lambda i: (i, 0))] * 4,
        out_shape=[jax.ShapeDtypeStruct((NG, 32 * G), _f32)] * 4,
    )(cf_g, vf_g, wce, bce, wve, bve, wj1a, wj1b)


def _tc_mlp_body(xa, xb, b1, wj2, b2, h2_o):
    h1 = jnp.maximum(xa[...] + xb[...] + b1[...][None, :], 0.0)
    h2_o[...] = jnp.maximum(_dot(h1, wj2[...]) + b2[...][None, :], 0.0)


def _tc_mlp(ha_g, hb_g, b1, wj2, b2):
    return pl.pallas_call(
        _tc_mlp_body,
        grid=(EHG // BE,),
        in_specs=[
            pl.BlockSpec((BE, 32 * G), lambda i: (i, 0)),
            pl.BlockSpec((BE, 32 * G), lambda i: (i, 0)),
            _full((32 * G,)), _full((32 * G, 32 * G)), _full((32 * G,)),
        ],
        out_specs=pl.BlockSpec((BE, 32 * G), lambda i: (i, 0)),
        out_shape=jax.ShapeDtypeStruct((EHG, 32 * G), _f32),
    )(ha_g, hb_g, b1, wj2, b2)


def _tc_update_body(agg0, agg1, c, wagg, wc, bcr, wj1a, a2_o):
    agg = agg0[...] + agg1[...]
    x = _dot(agg, wagg[...]) + _dot(c[...], wc[...])
    x = jnp.maximum(x + bcr[...][None, :], 0.0)
    a2_o[...] = _dot(x, wj1a[...])


def _tc_update(agg0_g, agg1_g, c_g, wagg, wc, bcr, wj1a):
    return pl.pallas_call(
        _tc_update_body,
        grid=(NG // BN,),
        in_specs=[
            pl.BlockSpec((BN, 32 * G), lambda i: (i, 0)),
            pl.BlockSpec((BN, 32 * G), lambda i: (i, 0)),
            pl.BlockSpec((BN, 32 * G), lambda i: (i, 0)),
            _full((32 * G, 32 * G)), _full((32 * G, 32 * G)),
            _full((32 * G,)), _full((32 * G, 32 * G)),
        ],
        out_specs=pl.BlockSpec((BN, 32 * G), lambda i: (i, 0)),
        out_shape=jax.ShapeDtypeStruct((NG, 32 * G), _f32),
    )(agg0_g, agg1_g, c_g, wagg, wc, bcr, wj1a)


def _tc_out_body(agg0, agg1, v, wagg, wv, bvr, wo1, bo1, wo2, bo2,
                 wo3, bo3, o):
    agg = agg0[...] + agg1[...]
    x = _dot(agg, wagg[...]) + _dot(v[...], wv[...])
    x = jnp.maximum(x + bvr[...][None, :], 0.0)
    x = jnp.maximum(_dot(x, wo1[...]) + bo1[...][None, :], 0.0)
    x = jnp.maximum(_dot(x, wo2[...]) + bo2[...][None, :], 0.0)
    o[...] = _dot(x, wo3[...]) + bo3[...][None, :]


def _tc_out(agg0_g, agg1_g, v_g, wagg, wv, bvr, wo1, bo1, wo2, bo2,
            wo3, bo3):
    return pl.pallas_call(
        _tc_out_body,
        grid=(NG // BN,),
        in_specs=[
            pl.BlockSpec((BN, 32 * G), lambda i: (i, 0)),
            pl.BlockSpec((BN, 32 * G), lambda i: (i, 0)),
            pl.BlockSpec((BN, 32 * G), lambda i: (i, 0)),
            _full((32 * G, 32 * G)), _full((32 * G, 32 * G)),
            _full((32 * G,)),
            _full((32 * G, 32 * G)), _full((32 * G,)),
            _full((32 * G, 32 * G)), _full((32 * G,)),
            _full((32 * G, G)), _full((G,)),
        ],
        out_specs=pl.BlockSpec((BN, G), lambda i: (i, 0)),
        out_shape=jax.ShapeDtypeStruct((NG, G), _f32),
    )(agg0_g, agg1_g, v_g, wagg, wv, bvr, wo1, bo1, wo2, bo2, wo3, bo3)


def kernel(cons_features, edge_indices, vars_features,
           W_ce, b_ce, W_ve, b_ve, W_j1, b_j1, W_j2, b_j2,
           W_cr, b_cr, W_vr, b_vr, W_o1, b_o1, W_o2, b_o2, W_o3, b_o3):
    src = edge_indices[0].astype(jnp.int32)
    dst = edge_indices[1].astype(jnp.int32)

    eye = jnp.eye(G, dtype=_f32)

    def bd(w):
        return jnp.kron(eye, w)

    def tile(b):
        return jnp.tile(b, G)

    def eg(x):
        return x.reshape(EHG, 32 * G)

    cf_g = cons_features.reshape(NG, 2 * G)
    vf_g = vars_features.reshape(NG, 9 * G)

    c_g, v_g, a1_g, b_g = _tc_pre(
        cf_g, vf_g, bd(W_ce), tile(b_ce), bd(W_ve), tile(b_ve),
        bd(W_j1[:EMB]), bd(W_j1[EMB:]))

    a1 = a1_g.reshape(N, EMB)
    b = b_g.reshape(N, EMB)

    wj2 = bd(W_j2)
    b1t = tile(b_j1)
    b2t = tile(b_j2)

    ha0, hb0 = _sc_gather2_h0(a1, b, src, dst)
    ha1, hb1 = _sc_gather2_h1(a1, b, src, dst)
    h2_0 = _tc_mlp(eg(ha0), eg(hb0), b1t, wj2, b2t)
    h2_1 = _tc_mlp(eg(ha1), eg(hb1), b1t, wj2, b2t)
    agg0 = _sc_scatter_h0(h2_0.reshape(EH, EMB), src)
    agg1 = _sc_scatter_h1(h2_1.reshape(EH, EMB), src)

    a2_g = _tc_update(agg0.reshape(NG, 32 * G), agg1.reshape(NG, 32 * G),
                      c_g, bd(W_cr[:EMB]), bd(W_cr[EMB:]),
                      tile(b_cr), bd(W_j1[:EMB]))
    a2 = a2_g.reshape(N, EMB)

    ha2_0 = _sc_gather1_h0(a2, src)
    ha2_1 = _sc_gather1_h1(a2, src)
    h2b_0 = _tc_mlp(eg(ha2_0), eg(hb0), b1t, wj2, b2t)
    h2b_1 = _tc_mlp(eg(ha2_1), eg(hb1), b1t, wj2, b2t)
    agg2_0 = _sc_scatter_h0(h2b_0.reshape(EH, EMB), dst)
    agg2_1 = _sc_scatter_h1(h2b_1.reshape(EH, EMB), dst)

    out_g = _tc_out(agg2_0.reshape(NG, 32 * G), agg2_1.reshape(NG, 32 * G),
                    v_g, bd(W_vr[:EMB]), bd(W_vr[EMB:]),
                    tile(b_vr), bd(W_o1), tile(b_o1), bd(W_o2), tile(b_o2),
                    bd(W_o3), tile(b_o3))
    return out_g.reshape(N, 1)


# speedup vs baseline: 13.3146x; 1.1598x over previous
import jax
import jax.numpy as jnp
from jax import lax
from jax.experimental import pallas as pl
from jax.experimental.pallas import tpu as pltpu
from jax.experimental.pallas import tpu_sc as plsc

N = 100000
E = 1600000
EMB = 32
H = 16
G = 4

NC = 2
NS = 16
NW = NC * NS

EW = E // NW
GCH = 1000
GNCH = EW // GCH

ES = E // NS
SCH = 800
SNCH = ES // SCH

NR = N // NS
ZN = NR // SCH
ZT = NR - ZN * SCH

NG = N // G
EG = E // G
BN = 5000
BE = 8000

_f32 = jnp.float32


def _sc_gather_body(a_hbm, b_hbm, src_hbm, dst_hbm, outa_hbm, outb_hbm,
                    idx_a, idx_b, buf_a, buf_b, gsem, wsem):
    cid = lax.axis_index("c")
    sid = lax.axis_index("s")
    wid = cid * NS + sid
    w0 = wid * EW

    def load_fire(k, s):
        base = pl.multiple_of(w0 + k * GCH, 8)
        pltpu.sync_copy(src_hbm.at[pl.ds(base, GCH)], idx_a)
        pltpu.sync_copy(dst_hbm.at[pl.ds(base, GCH)], idx_b)
        pltpu.async_copy(a_hbm.at[idx_a], buf_a[s], gsem[s])
        pltpu.async_copy(b_hbm.at[idx_b], buf_b[s], gsem[s])

    def wait_gather(k, s):
        pltpu.make_async_copy(a_hbm.at[idx_a], buf_a[s], gsem[s]).wait()
        pltpu.make_async_copy(b_hbm.at[idx_b], buf_b[s], gsem[s]).wait()

    def fire_write(k, s):
        base = pl.multiple_of(w0 + k * GCH, 8)
        pltpu.async_copy(buf_a[s], outa_hbm.at[pl.ds(base, GCH)], wsem[s])
        pltpu.async_copy(buf_b[s], outb_hbm.at[pl.ds(base, GCH)], wsem[s])

    def wait_write(k, s):
        base = pl.multiple_of(w0 + k * GCH, 8)
        pltpu.make_async_copy(buf_a[s], outa_hbm.at[pl.ds(base, GCH)],
                              wsem[s]).wait()
        pltpu.make_async_copy(buf_b[s], outb_hbm.at[pl.ds(base, GCH)],
                              wsem[s]).wait()

    load_fire(0, 0)

    def pair(p, carry):
        k = 2 * p
        wait_gather(k, 0)
        fire_write(k, 0)

        @pl.when(p > 0)
        def _():
            wait_write(k - 1, 1)

        load_fire(k + 1, 1)
        wait_gather(k + 1, 1)
        fire_write(k + 1, 1)
        wait_write(k, 0)

        @pl.when(p < GNCH // 2 - 1)
        def _():
            load_fire(k + 2, 0)

        return carry

    lax.fori_loop(0, GNCH // 2, pair, 0)
    wait_write(GNCH - 1, 1)


_sc_gather = pl.kernel(
    _sc_gather_body,
    out_type=(jax.ShapeDtypeStruct((E, EMB), _f32),
              jax.ShapeDtypeStruct((E, EMB), _f32)),
    compiler_params=pltpu.CompilerParams(use_tc_tiling_on_sc=False),
    mesh=plsc.VectorSubcoreMesh(core_axis_name="c", subcore_axis_name="s"),
    scratch_types=[
        pltpu.VMEM((GCH,), jnp.int32),
        pltpu.VMEM((GCH,), jnp.int32),
        [pltpu.VMEM((GCH, EMB), _f32)] * 2,
        [pltpu.VMEM((GCH, EMB), _f32)] * 2,
        [pltpu.SemaphoreType.DMA] * 2,
        [pltpu.SemaphoreType.DMA] * 2,
    ],
)


def _sc_gather1_body(a_hbm, idx_hbm, out_hbm, idx_a, buf_a, gsem, wsem):
    cid = lax.axis_index("c")
    sid = lax.axis_index("s")
    wid = cid * NS + sid
    w0 = wid * EW

    def load_fire(k, s):
        base = pl.multiple_of(w0 + k * GCH, 8)
        pltpu.sync_copy(idx_hbm.at[pl.ds(base, GCH)], idx_a)
        pltpu.async_copy(a_hbm.at[idx_a], buf_a[s], gsem[s])

    def wait_gather(k, s):
        pltpu.make_async_copy(a_hbm.at[idx_a], buf_a[s], gsem[s]).wait()

    def fire_write(k, s):
        base = pl.multiple_of(w0 + k * GCH, 8)
        pltpu.async_copy(buf_a[s], out_hbm.at[pl.ds(base, GCH)], wsem[s])

    def wait_write(k, s):
        base = pl.multiple_of(w0 + k * GCH, 8)
        pltpu.make_async_copy(buf_a[s], out_hbm.at[pl.ds(base, GCH)],
                              wsem[s]).wait()

    load_fire(0, 0)

    def pair(p, carry):
        k = 2 * p
        wait_gather(k, 0)
        fire_write(k, 0)

        @pl.when(p > 0)
        def _():
            wait_write(k - 1, 1)

        load_fire(k + 1, 1)
        wait_gather(k + 1, 1)
        fire_write(k + 1, 1)
        wait_write(k, 0)

        @pl.when(p < GNCH // 2 - 1)
        def _():
            load_fire(k + 2, 0)

        return carry

    lax.fori_loop(0, GNCH // 2, pair, 0)
    wait_write(GNCH - 1, 1)


_sc_gather1 = pl.kernel(
    _sc_gather1_body,
    out_type=jax.ShapeDtypeStruct((E, EMB), _f32),
    compiler_params=pltpu.CompilerParams(use_tc_tiling_on_sc=False),
    mesh=plsc.VectorSubcoreMesh(core_axis_name="c", subcore_axis_name="s"),
    scratch_types=[
        pltpu.VMEM((GCH,), jnp.int32),
        [pltpu.VMEM((GCH, EMB), _f32)] * 2,
        [pltpu.SemaphoreType.DMA] * 2,
        [pltpu.SemaphoreType.DMA] * 2,
    ],
)


def _sc_scatter_body(h2_hbm, idx_hbm, out_hbm, idx_v, rows_v, rsem, acc):
    cid = lax.axis_index("c")
    sid = lax.axis_index("s")
    s0 = sid * ES

    def zrow(i, c2):
        rows_v[0][i, :] = jnp.zeros((16,), _f32)
        return c2

    lax.fori_loop(0, SCH, zrow, 0)

    def zcp(j, c2):
        pltpu.sync_copy(rows_v[0], acc.at[pl.ds(sid * NR + j * SCH, SCH)])
        return c2

    lax.fori_loop(0, ZN, zcp, 0)
    pltpu.sync_copy(rows_v[0].at[pl.ds(0, ZT)],
                    acc.at[pl.ds(sid * NR + ZN * SCH, ZT)])
    plsc.subcore_barrier()

    def run_chunks(col):
        def load_fire(k, s):
            base = pl.multiple_of(s0 + k * SCH, 8)
            pltpu.sync_copy(idx_hbm.at[pl.ds(base, SCH)], idx_v[s])
            pltpu.async_copy(h2_hbm.at[pl.ds(base, SCH), pl.ds(col, H)],
                             rows_v[s], rsem[s])

        def wait_read(k, s):
            base = pl.multiple_of(s0 + k * SCH, 8)
            pltpu.make_async_copy(
                h2_hbm.at[pl.ds(base, SCH), pl.ds(col, H)],
                rows_v[s], rsem[s]).wait()

        load_fire(0, 0)

        def pair(p, carry):
            k = 2 * p
            wait_read(k, 0)
            load_fire(k + 1, 1)
            pltpu.sync_copy(rows_v[0], acc.at[idx_v[0]], add=True)
            wait_read(k + 1, 1)
            load_fire(k + 2, 0)
            pltpu.sync_copy(rows_v[1], acc.at[idx_v[1]], add=True)
            return carry

        lax.fori_loop(0, SNCH // 2, pair, 0)
        wait_read(SNCH - 1, 0)
        pltpu.sync_copy(rows_v[0], acc.at[idx_v[0]], add=True)

    @pl.when(cid == 0)
    def _():
        run_chunks(0)

    @pl.when(cid == 1)
    def _():
        run_chunks(H)

    plsc.subcore_barrier()
    r0 = sid * NR

    @pl.when(cid == 0)
    def _():
        pltpu.sync_copy(acc.at[pl.ds(r0, NR)],
                        out_hbm.at[pl.ds(r0, NR), pl.ds(0, H)])

    @pl.when(cid == 1)
    def _():
        pltpu.sync_copy(acc.at[pl.ds(r0, NR)],
                        out_hbm.at[pl.ds(r0, NR), pl.ds(H, H)])


_sc_scatter = pl.kernel(
    _sc_scatter_body,
    out_type=jax.ShapeDtypeStruct((N, EMB), _f32),
    compiler_params=pltpu.CompilerParams(use_tc_tiling_on_sc=False),
    mesh=plsc.VectorSubcoreMesh(core_axis_name="c", subcore_axis_name="s"),
    scratch_types=[
        [pltpu.VMEM((SCH,), jnp.int32)] * 2,
        [pltpu.VMEM((SCH, H), _f32)] * 2,
        [pltpu.SemaphoreType.DMA] * 2,
        pltpu.VMEM_SHARED((N, H), _f32),
    ],
)


def _full(shape):
    return pl.BlockSpec(shape, lambda i: tuple(0 for _ in shape))


def _dot(a, b):
    return jnp.dot(a, b, preferred_element_type=_f32)


def _tc_pre_body(cf, vf, wce, bce, wve, bve, wj1a, wj1b,
                 c_o, v_o, a_o, b_o):
    c = jnp.maximum(_dot(cf[...], wce[...]) + bce[...][None, :], 0.0)
    v = jnp.maximum(_dot(vf[...], wve[...]) + bve[...][None, :], 0.0)
    c_o[...] = c
    v_o[...] = v
    a_o[...] = _dot(c, wj1a[...])
    b_o[...] = _dot(v, wj1b[...])


def _tc_pre(cf_g, vf_g, wce, bce, wve, bve, wj1a, wj1b):
    return pl.pallas_call(
        _tc_pre_body,
        grid=(NG // BN,),
        in_specs=[
            pl.BlockSpec((BN, 2 * G), lambda i: (i, 0)),
            pl.BlockSpec((BN, 9 * G), lambda i: (i, 0)),
            _full((2 * G, 32 * G)), _full((32 * G,)),
            _full((9 * G, 32 * G)), _full((32 * G,)),
            _full((32 * G, 32 * G)), _full((32 * G, 32 * G)),
        ],
        out_specs=[pl.BlockSpec((BN, 32 * G), lambda i: (i, 0))] * 4,
        out_shape=[jax.ShapeDtypeStruct((NG, 32 * G), _f32)] * 4,
    )(cf_g, vf_g, wce, bce, wve, bve, wj1a, wj1b)


def _tc_mlp_body(xa, xb, b1, wj2, b2, h2_o):
    h1 = jnp.maximum(xa[...] + xb[...] + b1[...][None, :], 0.0)
    h2_o[...] = jnp.maximum(_dot(h1, wj2[...]) + b2[...][None, :], 0.0)


def _tc_mlp(ha_g, hb_g, b1, wj2, b2):
    return pl.pallas_call(
        _tc_mlp_body,
        grid=(EG // BE,),
        in_specs=[
            pl.BlockSpec((BE, 32 * G), lambda i: (i, 0)),
            pl.BlockSpec((BE, 32 * G), lambda i: (i, 0)),
            _full((32 * G,)), _full((32 * G, 32 * G)), _full((32 * G,)),
        ],
        out_specs=pl.BlockSpec((BE, 32 * G), lambda i: (i, 0)),
        out_shape=jax.ShapeDtypeStruct((EG, 32 * G), _f32),
    )(ha_g, hb_g, b1, wj2, b2)


def _tc_update_body(agg, c, wagg, wc, bcr, wj1a, a2_o):
    x = _dot(agg[...], wagg[...]) + _dot(c[...], wc[...])
    x = jnp.maximum(x + bcr[...][None, :], 0.0)
    a2_o[...] = _dot(x, wj1a[...])


def _tc_update(agg_g, c_g, wagg, wc, bcr, wj1a):
    return pl.pallas_call(
        _tc_update_body,
        grid=(NG // BN,),
        in_specs=[
            pl.BlockSpec((BN, 32 * G), lambda i: (i, 0)),
            pl.BlockSpec((BN, 32 * G), lambda i: (i, 0)),
            _full((32 * G, 32 * G)), _full((32 * G, 32 * G)),
            _full((32 * G,)), _full((32 * G, 32 * G)),
        ],
        out_specs=pl.BlockSpec((BN, 32 * G), lambda i: (i, 0)),
        out_shape=jax.ShapeDtypeStruct((NG, 32 * G), _f32),
    )(agg_g, c_g, wagg, wc, bcr, wj1a)


def _tc_out_body(agg, v, wagg, wv, bvr, wo1, bo1, wo2, bo2, wo3, bo3, o):
    x = _dot(agg[...], wagg[...]) + _dot(v[...], wv[...])
    x = jnp.maximum(x + bvr[...][None, :], 0.0)
    x = jnp.maximum(_dot(x, wo1[...]) + bo1[...][None, :], 0.0)
    x = jnp.maximum(_dot(x, wo2[...]) + bo2[...][None, :], 0.0)
    o[...] = _dot(x, wo3[...]) + bo3[...][None, :]


def _tc_out(agg_g, v_g, wagg, wv, bvr, wo1, bo1, wo2, bo2, wo3, bo3):
    return pl.pallas_call(
        _tc_out_body,
        grid=(NG // BN,),
        in_specs=[
            pl.BlockSpec((BN, 32 * G), lambda i: (i, 0)),
            pl.BlockSpec((BN, 32 * G), lambda i: (i, 0)),
            _full((32 * G, 32 * G)), _full((32 * G, 32 * G)),
            _full((32 * G,)),
            _full((32 * G, 32 * G)), _full((32 * G,)),
            _full((32 * G, 32 * G)), _full((32 * G,)),
            _full((32 * G, G)), _full((G,)),
        ],
        out_specs=pl.BlockSpec((BN, G), lambda i: (i, 0)),
        out_shape=jax.ShapeDtypeStruct((NG, G), _f32),
    )(agg_g, v_g, wagg, wv, bvr, wo1, bo1, wo2, bo2, wo3, bo3)


def kernel(cons_features, edge_indices, vars_features,
           W_ce, b_ce, W_ve, b_ve, W_j1, b_j1, W_j2, b_j2,
           W_cr, b_cr, W_vr, b_vr, W_o1, b_o1, W_o2, b_o2, W_o3, b_o3):
    src = edge_indices[0].astype(jnp.int32)
    dst = edge_indices[1].astype(jnp.int32)

    eye = jnp.eye(G, dtype=_f32)

    def bd(w):
        return jnp.kron(eye, w)

    def tile(b):
        return jnp.tile(b, G)

    cf_g = cons_features.reshape(NG, 2 * G)
    vf_g = vars_features.reshape(NG, 9 * G)

    c_g, v_g, a1_g, b_g = _tc_pre(
        cf_g, vf_g, bd(W_ce), tile(b_ce), bd(W_ve), tile(b_ve),
        bd(W_j1[:EMB]), bd(W_j1[EMB:]))

    a1 = a1_g.reshape(N, EMB)
    b = b_g.reshape(N, EMB)

    wj2 = bd(W_j2)
    b1t = tile(b_j1)
    b2t = tile(b_j2)

    ha, hb = _sc_gather(a1, b, src, dst)
    h2_g = _tc_mlp(ha.reshape(EG, 32 * G), hb.reshape(EG, 32 * G),
                   b1t, wj2, b2t)
    agg = _sc_scatter(h2_g.reshape(E, EMB), src)

    a2_g = _tc_update(agg.reshape(NG, 32 * G), c_g,
                      bd(W_cr[:EMB]), bd(W_cr[EMB:]),
                      tile(b_cr), bd(W_j1[:EMB]))

    ha2 = _sc_gather1(a2_g.reshape(N, EMB), src)
    h2b_g = _tc_mlp(ha2.reshape(EG, 32 * G), hb.reshape(EG, 32 * G),
                    b1t, wj2, b2t)
    agg2 = _sc_scatter(h2b_g.reshape(E, EMB), dst)

    out_g = _tc_out(agg2.reshape(NG, 32 * G), v_g,
                    bd(W_vr[:EMB]), bd(W_vr[EMB:]),
                    tile(b_vr), bd(W_o1), tile(b_o1), bd(W_o2), tile(b_o2),
                    bd(W_o3), tile(b_o3))
    return out_g.reshape(N, 1)
